# baseline (speedup 1.0000x reference)
import jax
import jax.numpy as jnp
from jax.experimental import pallas as pl

N = 1024


def _fused_kernel(a_ref, x_ref,
                  w1_ref, b1_ref, w2_ref, b2_ref, w3_ref, b3_ref,
                  wg1_ref, bg1_ref, wg2_ref, bg2_ref, wg3_ref, bg3_ref,
                  wg4_ref, bg4_ref, wg5_ref, bg5_ref, wg6_ref, bg6_ref,
                  o_ref):
    a = a_ref[...]
    row = jax.lax.broadcasted_iota(jnp.int32, (N, N), 0)
    col = jax.lax.broadcasted_iota(jnp.int32, (N, N), 1)
    aeff = jnp.where((a != 0) | (row == col), 1.0, 0.0).astype(jnp.bfloat16)

    def mm_t(lhs, rhs):
        return jax.lax.dot_general(lhs, rhs, (((0,), (0,)), ((), ())),
                                   preferred_element_type=jnp.float32)

    def mm(lhs, rhs):
        return jax.lax.dot_general(lhs, rhs, (((1,), (0,)), ((), ())),
                                   preferred_element_type=jnp.float32)

    bf = lambda v: v.astype(jnp.bfloat16)

    deg = mm_t(aeff, jnp.ones((N, 1), jnp.bfloat16))
    dinv = jnp.where(deg > 0, jax.lax.rsqrt(deg), 0.0)
    dinv_h, dinv_q = 0.5 * dinv, 0.25 * dinv

    def gcn(h, w_ref, b_ref, dout, bscale):
        q = bf(dinv * mm(h, w_ref[...]))
        return dout * mm_t(aeff, q) + bscale * b_ref[...].reshape(1, -1)

    relu = lambda v: jnp.maximum(v, 0.0)

    x1l = mm(x_ref[...], w1_ref[...]) + b1_ref[...].reshape(1, -1)
    x1 = x1l + relu(gcn(x1l, wg1_ref, bg1_ref, dinv, 1.0))
    x2l = mm(x1, w2_ref[...]) + b2_ref[...].reshape(1, -1)
    x2 = x2l + relu(gcn(x2l, wg2_ref, bg2_ref, dinv, 1.0))
    x3l = mm(x2, w3_ref[...]) + b3_ref[...].reshape(1, -1)
    x3 = x3l + relu(gcn(x3l, wg3_ref, bg3_ref, dinv_h, 0.5))
    x4 = x3 + relu(gcn(x3, wg4_ref, bg4_ref, dinv_h, 0.5))
    x5 = x4 + relu(gcn(x4, wg5_ref, bg5_ref, dinv_q, 0.25))
    x6 = x5 + gcn(x5, wg6_ref, bg6_ref, dinv_q, 0.25)
    o_ref[...] = x6


def kernel(x, A_pos, W1, b1, W2, b2, W3, b3, Wg1, bg1, Wg2, bg2, Wg3, bg3,
           Wg4, bg4, Wg5, bg5, Wg6, bg6):
    out = pl.pallas_call(
        _fused_kernel,
        out_shape=jax.ShapeDtypeStruct((N, 128), jnp.float32),
    )(A_pos, x, W1, b1, W2, b2, W3, b3, Wg1, bg1, Wg2, bg2, Wg3, bg3,
      Wg4, bg4, Wg5, bg5, Wg6, bg6)
    return out


# speedup vs baseline: 1.1989x; 1.1989x over previous
import jax
import jax.numpy as jnp
from jax.experimental import pallas as pl

N = 1024


def _fused_kernel(a_ref, x_ref,
                  w1_ref, b1_ref, w2_ref, b2_ref, w3_ref, b3_ref,
                  wg1_ref, bg1_ref, wg2_ref, bg2_ref, wg3_ref, bg3_ref,
                  wg4_ref, bg4_ref, wg5_ref, bg5_ref, wg6_ref, bg6_ref,
                  o_ref):
    a = a_ref[...]
    row = jax.lax.broadcasted_iota(jnp.int32, (N, N), 0)
    col = jax.lax.broadcasted_iota(jnp.int32, (N, N), 1)
    aeff = jnp.where((a != 0) | (row == col), 1.0, 0.0).astype(jnp.bfloat16)

    def mm_t(lhs, rhs):
        return jax.lax.dot_general(lhs, rhs, (((0,), (0,)), ((), ())),
                                   preferred_element_type=jnp.float32)

    def mm(lhs, rhs):
        return jax.lax.dot_general(lhs, rhs, (((1,), (0,)), ((), ())),
                                   preferred_element_type=jnp.float32)

    bf = lambda v: v.astype(jnp.bfloat16)

    deg = mm_t(aeff, jnp.ones((N, 1), jnp.bfloat16))
    dinv = jnp.where(deg > 0, jax.lax.rsqrt(deg), 0.0)
    dinv_h, dinv_q = 0.5 * dinv, 0.25 * dinv

    def gcn(h, w_ref, b_ref, dout, bscale):
        q = bf(dinv * mm(h, w_ref[...]))
        z = jnp.concatenate(
            [mm_t(aeff[:, : N // 2], q), mm_t(aeff[:, N // 2:], q)], axis=0)
        return dout * z + bscale * b_ref[...].reshape(1, -1)

    relu = lambda v: jnp.maximum(v, 0.0)

    x1l = mm(x_ref[...], w1_ref[...]) + b1_ref[...].reshape(1, -1)
    x1 = x1l + relu(gcn(x1l, wg1_ref, bg1_ref, dinv, 1.0))
    x2l = mm(x1, w2_ref[...]) + b2_ref[...].reshape(1, -1)
    x2 = x2l + relu(gcn(x2l, wg2_ref, bg2_ref, dinv, 1.0))
    x3l = mm(x2, w3_ref[...]) + b3_ref[...].reshape(1, -1)
    x3 = x3l + relu(gcn(x3l, wg3_ref, bg3_ref, dinv_h, 0.5))
    x4 = x3 + relu(gcn(x3, wg4_ref, bg4_ref, dinv_h, 0.5))
    x5 = x4 + relu(gcn(x4, wg5_ref, bg5_ref, dinv_q, 0.25))
    x6 = x5 + gcn(x5, wg6_ref, bg6_ref, dinv_q, 0.25)
    o_ref[...] = x6


def kernel(x, A_pos, W1, b1, W2, b2, W3, b3, Wg1, bg1, Wg2, bg2, Wg3, bg3,
           Wg4, bg4, Wg5, bg5, Wg6, bg6):
    out = pl.pallas_call(
        _fused_kernel,
        out_shape=jax.ShapeDtypeStruct((N, 128), jnp.float32),
    )(A_pos, x, W1, b1, W2, b2, W3, b3, Wg1, bg1, Wg2, bg2, Wg3, bg3,
      Wg4, bg4, Wg5, bg5, Wg6, bg6)
    return out


# speedup vs baseline: 1.2868x; 1.0733x over previous
import jax
import jax.numpy as jnp
from jax.experimental import pallas as pl

N = 1024
H = N // 2


def _fused_kernel(a_ref, x_ref,
                  w1_ref, b1_ref, w2_ref, b2_ref, w3_ref, b3_ref,
                  wg1_ref, bg1_ref, wg2_ref, bg2_ref, wg3_ref, bg3_ref,
                  wg4_ref, bg4_ref, wg5_ref, bg5_ref, wg6_ref, bg6_ref,
                  o_ref):
    row = jax.lax.broadcasted_iota(jnp.int32, (H, H), 0)
    col = jax.lax.broadcasted_iota(jnp.int32, (H, H), 1)
    diag = row == col
    a11 = jnp.where((a_ref[:H, :H] != 0) | diag, 1.0, 0.0).astype(jnp.bfloat16)
    a22 = jnp.where((a_ref[H:, H:] != 0) | diag, 1.0, 0.0).astype(jnp.bfloat16)
    a12 = jnp.where(a_ref[:H, H:] != 0, 1.0, 0.0).astype(jnp.bfloat16)
    a21 = jnp.where(a_ref[H:, :H] != 0, 1.0, 0.0).astype(jnp.bfloat16)

    def mm_t(lhs, rhs):
        return jax.lax.dot_general(lhs, rhs, (((0,), (0,)), ((), ())),
                                   preferred_element_type=jnp.float32)

    def mm(lhs, rhs):
        return jax.lax.dot_general(lhs, rhs, (((1,), (0,)), ((), ())),
                                   preferred_element_type=jnp.float32)

    bf = lambda v: v.astype(jnp.bfloat16)
    ones = jnp.ones((H, 1), jnp.bfloat16)

    deg_a = mm_t(a11, ones) + mm_t(a21, ones)
    deg_b = mm_t(a12, ones) + mm_t(a22, ones)
    dinv_a = jnp.where(deg_a > 0, jax.lax.rsqrt(deg_a), 0.0)
    dinv_b = jnp.where(deg_b > 0, jax.lax.rsqrt(deg_b), 0.0)

    def gcn(ha, hb, w_ref, b_ref, oscale, bscale):
        w = w_ref[...]
        qa = bf(dinv_a * mm(ha, w))
        qb = bf(dinv_b * mm(hb, w))
        za = mm_t(a11, qa) + mm_t(a21, qb)
        zb = mm_t(a12, qa) + mm_t(a22, qb)
        b = bscale * b_ref[...].reshape(1, -1)
        return (oscale * dinv_a) * za + b, (oscale * dinv_b) * zb + b

    relu = lambda v: jnp.maximum(v, 0.0)

    def layer(ha, hb, wl_ref, bl_ref, wg_ref, bg_ref, oscale):
        la = mm(ha, wl_ref[...]) + bl_ref[...].reshape(1, -1)
        lb = mm(hb, wl_ref[...]) + bl_ref[...].reshape(1, -1)
        ga, gb = gcn(la, lb, wg_ref, bg_ref, oscale, oscale)
        return la + relu(ga), lb + relu(gb)

    x1a, x1b = layer(x_ref[:H], x_ref[H:], w1_ref, b1_ref, wg1_ref, bg1_ref, 1.0)
    x2a, x2b = layer(x1a, x1b, w2_ref, b2_ref, wg2_ref, bg2_ref, 1.0)
    x3a, x3b = layer(x2a, x2b, w3_ref, b3_ref, wg3_ref, bg3_ref, 0.5)
    g4a, g4b = gcn(x3a, x3b, wg4_ref, bg4_ref, 0.5, 0.5)
    x4a, x4b = x3a + relu(g4a), x3b + relu(g4b)
    g5a, g5b = gcn(x4a, x4b, wg5_ref, bg5_ref, 0.25, 0.25)
    x5a, x5b = x4a + relu(g5a), x4b + relu(g5b)
    g6a, g6b = gcn(x5a, x5b, wg6_ref, bg6_ref, 0.25, 0.25)
    o_ref[:H] = x5a + g6a
    o_ref[H:] = x5b + g6b


def kernel(x, A_pos, W1, b1, W2, b2, W3, b3, Wg1, bg1, Wg2, bg2, Wg3, bg3,
           Wg4, bg4, Wg5, bg5, Wg6, bg6):
    out = pl.pallas_call(
        _fused_kernel,
        out_shape=jax.ShapeDtypeStruct((N, 128), jnp.float32),
    )(A_pos, x, W1, b1, W2, b2, W3, b3, Wg1, bg1, Wg2, bg2, Wg3, bg3,
      Wg4, bg4, Wg5, bg5, Wg6, bg6)
    return out
